# baseline (speedup 1.0000x reference)
import functools

import jax
import jax.numpy as jnp
from jax import lax
from jax.experimental import pallas as pl
from jax.experimental.pallas import tpu as pltpu
from jax.experimental.pallas import tpu_sc as plsc

B, S, EMB = 4, 2048, 128
NW = 32
RPW = (B * S) // NW
CH = EMB // 16
NG = RPW // 16
PAD = 17
GPAD = 16 * PAD


def _body(word_hbm, idx_hbm, pos_hbm, aux_hbm, out_hbm,
          idx_v, rows_v, pos_v, aux_v, x_v, ssum_v, ssq_v,
          sstat_v, mstat_v, sem, osem):
    wid = lax.axis_index("s") * 2 + lax.axis_index("c")
    base = wid * RPW
    sbase = (wid % (S // RPW)) * RPW

    pltpu.sync_copy(idx_hbm.at[pl.ds(base, RPW)], idx_v)
    cp0 = pltpu.async_copy(word_hbm.at[idx_v.at[pl.ds(0, 128)]],
                           rows_v.at[pl.ds(0, 128)], sem)
    cp1 = pltpu.async_copy(word_hbm.at[idx_v.at[pl.ds(128, 128)]],
                           rows_v.at[pl.ds(128, 128)], sem)
    pltpu.sync_copy(pos_hbm.at[pl.ds(sbase, RPW)], pos_v)
    pltpu.sync_copy(aux_hbm, aux_v)
    cp0.wait()

    inv_n = jnp.float32(1.0 / EMB)
    lane = lax.iota(jnp.int32, 16)
    colbase = lane * PAD
    zero16 = lane * 0
    tch = [aux_v[pl.ds(16 * c, 16)] for c in range(CH)]
    gch = [aux_v[pl.ds(EMB + 16 * c, 16)] for c in range(CH)]
    bch = [aux_v[pl.ds(2 * EMB + 16 * c, 16)] for c in range(CH)]

    def group_body(g):
        r0 = g * 16
        sb = g * GPAD
        tb = g * 16
        for rr in range(16):
            r = r0 + rr
            s = None
            sq = None
            for c in range(CH):
                d = pl.ds(16 * c, 16)
                x = (rows_v[r, d] + pos_v[r, d]) + tch[c]
                x_v[r, d] = x
                xx = x * x
                s = x if s is None else s + x
                sq = xx if sq is None else sq + xx
            ssum_v[pl.ds(sb + PAD * rr, 16)] = s
            ssq_v[pl.ds(sb + PAD * rr, 16)] = sq
        tot = None
        totsq = None
        for j in range(16):
            idxj = colbase + (sb + j)
            ts = plsc.load_gather(ssum_v, [idxj])
            tq = plsc.load_gather(ssq_v, [idxj])
            tot = ts if tot is None else tot + ts
            totsq = tq if totsq is None else totsq + tq
        mean = tot * inv_n
        var = totsq * inv_n - mean * mean
        v = var + jnp.float32(1e-5)
        i = plsc.bitcast(v, jnp.int32)
        i = jnp.int32(0x5F3759DF) - (i >> 1)
        y = plsc.bitcast(i, jnp.float32)
        half_v = v * jnp.float32(0.5)
        y = y * (jnp.float32(1.5) - half_v * y * y)
        y = y * (jnp.float32(1.5) - half_v * y * y)
        y = y * (jnp.float32(1.5) - half_v * y * y)
        sstat_v[pl.ds(tb, 16)] = y
        mstat_v[pl.ds(tb, 16)] = mean * y
        for rr in range(16):
            r = r0 + rr
            bidx = zero16 + (tb + rr)
            sc = plsc.load_gather(sstat_v, [bidx])
            ms = plsc.load_gather(mstat_v, [bidx])
            for c in range(CH):
                d = pl.ds(16 * c, 16)
                t = x_v[r, d] * sc - ms
                rows_v[r, d] = t * gch[c] + bch[c]

    @plsc.parallel_loop(0, NG // 2)
    def group1(g):
        group_body(g)

    out0 = pltpu.async_copy(rows_v.at[pl.ds(0, 128)],
                            out_hbm.at[pl.ds(base, 128)], osem)
    cp1.wait()

    @plsc.parallel_loop(NG // 2, NG)
    def group2(g):
        group_body(g)

    out1 = pltpu.async_copy(rows_v.at[pl.ds(128, 128)],
                            out_hbm.at[pl.ds(base + 128, 128)], osem)
    out0.wait()
    out1.wait()


@jax.jit
def _run(word_emb, idx, pos_sl, aux):
    mesh = plsc.VectorSubcoreMesh(core_axis_name="c", subcore_axis_name="s")
    k = functools.partial(
        pl.kernel,
        mesh=mesh,
        compiler_params=pltpu.CompilerParams(needs_layout_passes=False),
        out_type=jax.ShapeDtypeStruct((B * S, EMB), jnp.float32),
        scratch_types=[
            pltpu.VMEM((RPW,), jnp.int32),
            pltpu.VMEM((RPW, EMB), jnp.float32),
            pltpu.VMEM((RPW, EMB), jnp.float32),
            pltpu.VMEM((3 * EMB,), jnp.float32),
            pltpu.VMEM((RPW, EMB), jnp.float32),
            pltpu.VMEM((NG * GPAD,), jnp.float32),
            pltpu.VMEM((NG * GPAD,), jnp.float32),
            pltpu.VMEM((NG * 16,), jnp.float32),
            pltpu.VMEM((NG * 16,), jnp.float32),
            pltpu.SemaphoreType.DMA,
            pltpu.SemaphoreType.DMA,
        ],
    )(_body)
    return k(word_emb, idx, pos_sl, aux)


def kernel(input_ids, word_emb, pos_emb, type_emb, ln_gamma, ln_beta):
    idx = input_ids.astype(jnp.int32).reshape(B * S)
    pos_sl = pos_emb[2:2 + S]
    aux = jnp.concatenate([type_emb[0], ln_gamma, ln_beta])
    out = _run(word_emb, idx, pos_sl, aux)
    return out.reshape(B, S, EMB)


# speedup vs baseline: 1.0219x; 1.0219x over previous
import functools

import jax
import jax.numpy as jnp
from jax import lax
from jax.experimental import pallas as pl
from jax.experimental.pallas import tpu as pltpu
from jax.experimental.pallas import tpu_sc as plsc

B, S, EMB = 4, 2048, 128
NW = 32
RPW = (B * S) // NW
CH = EMB // 16
NG = RPW // 16
PAD = 17
GPAD = 16 * PAD


def _body(word_hbm, idx_hbm, pos_hbm, aux_hbm, out_hbm,
          idx_v, rows_v, pos_v, aux_v, x_v, ssum_v, ssq_v,
          sstat_v, mstat_v, sem, osem):
    wid = lax.axis_index("s") * 2 + lax.axis_index("c")
    base = wid * RPW
    sbase = (wid % (S // RPW)) * RPW

    pltpu.sync_copy(idx_hbm.at[pl.ds(base, RPW)], idx_v)
    cp0 = pltpu.async_copy(word_hbm.at[idx_v.at[pl.ds(0, 128)]],
                           rows_v.at[pl.ds(0, 128)], sem)
    cp1 = pltpu.async_copy(word_hbm.at[idx_v.at[pl.ds(128, 128)]],
                           rows_v.at[pl.ds(128, 128)], sem)
    pltpu.sync_copy(pos_hbm.at[pl.ds(sbase, RPW)], pos_v)
    pltpu.sync_copy(aux_hbm, aux_v)
    cp0.wait()
    cp1.wait()

    inv_n = jnp.float32(1.0 / EMB)
    lane = lax.iota(jnp.int32, 16)
    colbase = lane * PAD
    zero16 = lane * 0
    tch = [aux_v[pl.ds(16 * c, 16)] for c in range(CH)]
    gch = [aux_v[pl.ds(EMB + 16 * c, 16)] for c in range(CH)]
    bch = [aux_v[pl.ds(2 * EMB + 16 * c, 16)] for c in range(CH)]

    def group_body(g):
        r0 = g * 16
        sb = g * GPAD
        tb = g * 16
        for rr in range(16):
            r = r0 + rr
            s = None
            sq = None
            for c in range(CH):
                d = pl.ds(16 * c, 16)
                x = (rows_v[r, d] + pos_v[r, d]) + tch[c]
                x_v[r, d] = x
                xx = x * x
                s = x if s is None else s + x
                sq = xx if sq is None else sq + xx
            ssum_v[pl.ds(sb + PAD * rr, 16)] = s
            ssq_v[pl.ds(sb + PAD * rr, 16)] = sq
        tot = None
        totsq = None
        for j in range(16):
            idxj = colbase + (sb + j)
            ts = plsc.load_gather(ssum_v, [idxj])
            tq = plsc.load_gather(ssq_v, [idxj])
            tot = ts if tot is None else tot + ts
            totsq = tq if totsq is None else totsq + tq
        mean = tot * inv_n
        var = totsq * inv_n - mean * mean
        v = var + jnp.float32(1e-5)
        i = plsc.bitcast(v, jnp.int32)
        i = jnp.int32(0x5F3759DF) - (i >> 1)
        y = plsc.bitcast(i, jnp.float32)
        half_v = v * jnp.float32(0.5)
        y = y * (jnp.float32(1.5) - half_v * y * y)
        y = y * (jnp.float32(1.5) - half_v * y * y)
        y = y * (jnp.float32(1.5) - half_v * y * y)
        sstat_v[pl.ds(tb, 16)] = y
        mstat_v[pl.ds(tb, 16)] = mean * y
        for rr in range(16):
            r = r0 + rr
            bidx = zero16 + (tb + rr)
            sc = plsc.load_gather(sstat_v, [bidx])
            ms = plsc.load_gather(mstat_v, [bidx])
            for c in range(CH):
                d = pl.ds(16 * c, 16)
                t = x_v[r, d] * sc - ms
                rows_v[r, d] = t * gch[c] + bch[c]

    @plsc.parallel_loop(0, NG)
    def group1(g):
        group_body(g)

    pltpu.sync_copy(rows_v, out_hbm.at[pl.ds(base, RPW)])


@jax.jit
def _run(word_emb, idx, pos_sl, aux):
    mesh = plsc.VectorSubcoreMesh(core_axis_name="c", subcore_axis_name="s")
    k = functools.partial(
        pl.kernel,
        mesh=mesh,
        compiler_params=pltpu.CompilerParams(
            needs_layout_passes=False,
            skip_device_barrier=True,
            disable_bounds_checks=True,
            disable_semaphore_checks=True,
        ),
        out_type=jax.ShapeDtypeStruct((B * S, EMB), jnp.float32),
        scratch_types=[
            pltpu.VMEM((RPW,), jnp.int32),
            pltpu.VMEM((RPW, EMB), jnp.float32),
            pltpu.VMEM((RPW, EMB), jnp.float32),
            pltpu.VMEM((3 * EMB,), jnp.float32),
            pltpu.VMEM((RPW, EMB), jnp.float32),
            pltpu.VMEM((NG * GPAD,), jnp.float32),
            pltpu.VMEM((NG * GPAD,), jnp.float32),
            pltpu.VMEM((NG * 16,), jnp.float32),
            pltpu.VMEM((NG * 16,), jnp.float32),
            pltpu.SemaphoreType.DMA,
            pltpu.SemaphoreType.DMA,
        ],
    )(_body)
    return k(word_emb, idx, pos_sl, aux)


def kernel(input_ids, word_emb, pos_emb, type_emb, ln_gamma, ln_beta):
    idx = input_ids.astype(jnp.int32).reshape(B * S)
    pos_sl = pos_emb[2:2 + S]
    aux = jnp.concatenate([type_emb[0], ln_gamma, ln_beta])
    out = _run(word_emb, idx, pos_sl, aux)
    return out.reshape(B, S, EMB)


# speedup vs baseline: 1.0410x; 1.0187x over previous
import functools

import jax
import jax.numpy as jnp
from jax import lax
from jax.experimental import pallas as pl
from jax.experimental.pallas import tpu as pltpu
from jax.experimental.pallas import tpu_sc as plsc

B, S, EMB = 4, 2048, 128
NW = 32
RPW = (B * S) // NW
CH = EMB // 16
NG = RPW // 16
PAD = 17
GPAD = 16 * PAD


def _body(word_hbm, idx_hbm, pos_hbm, aux_hbm, out_hbm,
          idx_v, rows_v, pos_v, aux_v, x_v, ssum_v, ssq_v,
          sstat_v, mstat_v, sem, osem):
    wid = lax.axis_index("s") * 2 + lax.axis_index("c")
    base = wid * RPW
    sbase = (wid % (S // RPW)) * RPW

    pltpu.sync_copy(idx_hbm.at[pl.ds(base, RPW)], idx_v)
    for h in range(4):
        pltpu.async_copy(word_hbm.at[idx_v.at[pl.ds(64 * h, 64)]],
                         rows_v.at[pl.ds(64 * h, 64)], sem)
    pltpu.sync_copy(pos_hbm.at[pl.ds(sbase, RPW)], pos_v)
    pltpu.sync_copy(aux_hbm, aux_v)

    inv_n = jnp.float32(1.0 / EMB)
    lane = lax.iota(jnp.int32, 16)
    colbase = lane * PAD
    zero16 = lane * 0
    tch = [aux_v[pl.ds(16 * c, 16)] for c in range(CH)]
    gch = [aux_v[pl.ds(EMB + 16 * c, 16)] for c in range(CH)]
    bch = [aux_v[pl.ds(2 * EMB + 16 * c, 16)] for c in range(CH)]

    def group_body(g):
        r0 = g * 16
        sb = g * GPAD
        tb = g * 16
        for rr in range(16):
            r = r0 + rr
            s = None
            sq = None
            for c in range(CH):
                d = pl.ds(16 * c, 16)
                x = (rows_v[r, d] + pos_v[r, d]) + tch[c]
                x_v[r, d] = x
                xx = x * x
                s = x if s is None else s + x
                sq = xx if sq is None else sq + xx
            ssum_v[pl.ds(sb + PAD * rr, 16)] = s
            ssq_v[pl.ds(sb + PAD * rr, 16)] = sq
        tot = None
        totsq = None
        for j in range(16):
            idxj = colbase + (sb + j)
            ts = plsc.load_gather(ssum_v, [idxj])
            tq = plsc.load_gather(ssq_v, [idxj])
            tot = ts if tot is None else tot + ts
            totsq = tq if totsq is None else totsq + tq
        mean = tot * inv_n
        var = totsq * inv_n - mean * mean
        v = var + jnp.float32(1e-5)
        i = plsc.bitcast(v, jnp.int32)
        i = jnp.int32(0x5F3759DF) - (i >> 1)
        y = plsc.bitcast(i, jnp.float32)
        half_v = v * jnp.float32(0.5)
        y = y * (jnp.float32(1.5) - half_v * y * y)
        y = y * (jnp.float32(1.5) - half_v * y * y)
        y = y * (jnp.float32(1.5) - half_v * y * y)
        sstat_v[pl.ds(tb, 16)] = y
        mstat_v[pl.ds(tb, 16)] = mean * y
        for rr in range(16):
            r = r0 + rr
            bidx = zero16 + (tb + rr)
            sc = plsc.load_gather(sstat_v, [bidx])
            ms = plsc.load_gather(mstat_v, [bidx])
            for c in range(CH):
                d = pl.ds(16 * c, 16)
                t = x_v[r, d] * sc - ms
                rows_v[r, d] = t * gch[c] + bch[c]

    def chunk(h, carry):
        o = h * 64
        pltpu.make_async_copy(word_hbm.at[idx_v.at[pl.ds(o, 64)]],
                              rows_v.at[pl.ds(o, 64)], sem).wait()

        @plsc.parallel_loop(h * 4, h * 4 + 4)
        def group1(g):
            group_body(g)

        pltpu.async_copy(rows_v.at[pl.ds(o, 64)],
                         out_hbm.at[pl.ds(base + o, 64)], osem)
        return carry

    lax.fori_loop(0, 4, chunk, jnp.int32(0))
    for h in range(4):
        pltpu.make_async_copy(rows_v.at[pl.ds(64 * h, 64)],
                              out_hbm.at[pl.ds(base + 64 * h, 64)],
                              osem).wait()


@jax.jit
def _run(word_emb, idx, pos_sl, aux):
    mesh = plsc.VectorSubcoreMesh(core_axis_name="c", subcore_axis_name="s")
    k = functools.partial(
        pl.kernel,
        mesh=mesh,
        compiler_params=pltpu.CompilerParams(
            needs_layout_passes=False,
            skip_device_barrier=True,
            disable_bounds_checks=True,
            disable_semaphore_checks=True,
        ),
        out_type=jax.ShapeDtypeStruct((B * S, EMB), jnp.float32),
        scratch_types=[
            pltpu.VMEM((RPW,), jnp.int32),
            pltpu.VMEM((RPW, EMB), jnp.float32),
            pltpu.VMEM((RPW, EMB), jnp.float32),
            pltpu.VMEM((3 * EMB,), jnp.float32),
            pltpu.VMEM((RPW, EMB), jnp.float32),
            pltpu.VMEM((NG * GPAD,), jnp.float32),
            pltpu.VMEM((NG * GPAD,), jnp.float32),
            pltpu.VMEM((NG * 16,), jnp.float32),
            pltpu.VMEM((NG * 16,), jnp.float32),
            pltpu.SemaphoreType.DMA,
            pltpu.SemaphoreType.DMA,
        ],
    )(_body)
    return k(word_emb, idx, pos_sl, aux)


def kernel(input_ids, word_emb, pos_emb, type_emb, ln_gamma, ln_beta):
    idx = input_ids.astype(jnp.int32).reshape(B * S)
    pos_sl = pos_emb[2:2 + S]
    aux = jnp.concatenate([type_emb[0], ln_gamma, ln_beta])
    out = _run(word_emb, idx, pos_sl, aux)
    return out.reshape(B, S, EMB)
